# baseline (speedup 1.0000x reference)
import functools

import jax
import jax.numpy as jnp
from jax import lax
from jax.experimental import pallas as pl
from jax.experimental.pallas import tpu as pltpu

ATOM_FDIM = 128
BOND_FDIM = 144
HIDDEN = 256
DEPTH = 3
N_ATOMS = 10000
N_BONDS = 320000
MAX_NB = 32
N_MOLS = 200

BT = 2048


def _mm0_body(x_ref, w_ref, inp_ref, msg_ref):
    acc = jnp.dot(x_ref[...], w_ref[...], preferred_element_type=jnp.float32)
    inp_ref[...] = acc
    msg_ref[...] = jnp.maximum(acc, 0.0)


def _mm0(f_bonds, W_i):
    grid = (N_BONDS // BT,)
    return pl.pallas_call(
        _mm0_body,
        grid=grid,
        in_specs=[
            pl.BlockSpec((BT, BOND_FDIM), lambda i: (i, 0)),
            pl.BlockSpec((BOND_FDIM, HIDDEN), lambda i: (0, 0)),
        ],
        out_specs=[
            pl.BlockSpec((BT, HIDDEN), lambda i: (i, 0)),
            pl.BlockSpec((BT, HIDDEN), lambda i: (i, 0)),
        ],
        out_shape=[
            jax.ShapeDtypeStruct((N_BONDS, HIDDEN), jnp.float32),
            jax.ShapeDtypeStruct((N_BONDS, HIDDEN), jnp.float32),
        ],
    )(f_bonds, W_i)


def _mmh_body(pre_ref, w_ref, inp_ref, msg_ref):
    acc = jnp.dot(pre_ref[...], w_ref[...], preferred_element_type=jnp.float32)
    msg_ref[...] = jnp.maximum(inp_ref[...] + acc, 0.0)


def _mmh(msg_pre, W_h, inp):
    grid = (N_BONDS // BT,)
    return pl.pallas_call(
        _mmh_body,
        grid=grid,
        in_specs=[
            pl.BlockSpec((BT, HIDDEN), lambda i: (i, 0)),
            pl.BlockSpec((HIDDEN, HIDDEN), lambda i: (0, 0)),
            pl.BlockSpec((BT, HIDDEN), lambda i: (i, 0)),
        ],
        out_specs=pl.BlockSpec((BT, HIDDEN), lambda i: (i, 0)),
        out_shape=jax.ShapeDtypeStruct((N_BONDS, HIDDEN), jnp.float32),
    )(msg_pre, W_h, inp)


AT = 2000
MOLP = 256


def _readout_body(fa_ref, am_ref, wo1_ref, wo2_ref, bo_ref, mid_ref,
                  sums_ref, cnts_ref):
    i = pl.program_id(0)
    h = jnp.dot(fa_ref[...], wo1_ref[...], preferred_element_type=jnp.float32)
    h = h + jnp.dot(am_ref[...], wo2_ref[...], preferred_element_type=jnp.float32)
    h = jnp.maximum(h + bo_ref[...], 0.0)
    ids = mid_ref[...]
    onehot = (ids == lax.broadcasted_iota(jnp.int32, (AT, MOLP), 1)).astype(jnp.float32)
    part_sums = jnp.dot(onehot.T, h, preferred_element_type=jnp.float32)
    part_cnts = jnp.sum(onehot, axis=0, keepdims=True)

    @pl.when(i == 0)
    def _init():
        sums_ref[...] = jnp.zeros_like(sums_ref)
        cnts_ref[...] = jnp.zeros_like(cnts_ref)

    sums_ref[...] += part_sums
    cnts_ref[...] += part_cnts


def _readout(f_atoms, a_message, W_o, b_o, mol_ids):
    W_o1 = W_o[:ATOM_FDIM]
    W_o2 = W_o[ATOM_FDIM:]
    grid = (N_ATOMS // AT,)
    sums, cnts = pl.pallas_call(
        _readout_body,
        grid=grid,
        in_specs=[
            pl.BlockSpec((AT, ATOM_FDIM), lambda i: (i, 0)),
            pl.BlockSpec((AT, HIDDEN), lambda i: (i, 0)),
            pl.BlockSpec((ATOM_FDIM, HIDDEN), lambda i: (0, 0)),
            pl.BlockSpec((HIDDEN, HIDDEN), lambda i: (0, 0)),
            pl.BlockSpec((1, HIDDEN), lambda i: (0, 0)),
            pl.BlockSpec((AT, 1), lambda i: (i, 0)),
        ],
        out_specs=[
            pl.BlockSpec((MOLP, HIDDEN), lambda i: (0, 0)),
            pl.BlockSpec((1, MOLP), lambda i: (0, 0)),
        ],
        out_shape=[
            jax.ShapeDtypeStruct((MOLP, HIDDEN), jnp.float32),
            jax.ShapeDtypeStruct((1, MOLP), jnp.float32),
        ],
    )(f_atoms, a_message, W_o1, W_o2, b_o.reshape(1, HIDDEN),
      mol_ids.reshape(N_ATOMS, 1))
    mol_vecs = sums[:N_MOLS] / jnp.maximum(cnts[0, :N_MOLS], 1.0)[:, None]
    return mol_vecs


def _gather_phase(message, a2b, b2a, b2revb):
    nei = jnp.take(message, a2b, axis=0)
    a_message = nei.sum(axis=1)
    rev = jnp.take(message, b2revb, axis=0)
    msg_pre = jnp.take(a_message, b2a, axis=0) - rev
    return a_message, msg_pre


def kernel(f_atoms, f_bonds, W_i, W_h, W_o, b_o, a2b, b2a, b2revb, mol_ids):
    inp, message = _mm0(f_bonds, W_i)
    for _ in range(DEPTH - 1):
        _, msg_pre = _gather_phase(message, a2b, b2a, b2revb)
        message = _mmh(msg_pre, W_h, inp)
    nei = jnp.take(message, a2b, axis=0)
    a_message = nei.sum(axis=1)
    return _readout(f_atoms, a_message, W_o, b_o, mol_ids)
